# Initial kernel scaffold; baseline (speedup 1.0000x reference)
import functools

import jax
import jax.numpy as jnp
from jax import lax
from jax.experimental import pallas as pl
from jax.experimental.pallas import tpu as pltpu
from jax.experimental.pallas import tpu_sc as plsc

_B, _C, _H, _W = 8, 96, 224, 224
_N = 2000
_HW = _H * _W
_CHW = _C * _HW

_NUM_WORKERS = 32
_NPAD = 2048
_PTS = _NPAD // _NUM_WORKERS
_LANES = 16
_GCHUNK = 8

_BN = 40


def _sc_gather_kernel(inflat, ids_hbm, rows_hbm, cols_hbm, out_hbm,
                      ids_v, rows_v, cols_v, idx_v, g_v, sem):
    nc = lax.axis_size("c")
    wid = lax.axis_index("s") * nc + lax.axis_index("c")
    base = wid * _PTS

    pltpu.sync_copy(ids_hbm.at[pl.ds(base, _PTS)], ids_v)
    pltpu.sync_copy(rows_hbm.at[pl.ds(base, _PTS)], rows_v)
    pltpu.sync_copy(cols_hbm.at[pl.ds(base, _PTS)], cols_v)

    lane = lax.iota(jnp.int32, _LANES)
    bvecs = []
    for gr in range(_PTS // _LANES):
        sl = pl.ds(gr * _LANES, _LANES)
        bvecs.append(ids_v[sl] * _CHW + rows_v[sl] * _W + cols_v[sl])

    def build(c, carry):
        coff = c * _HW
        for gr in range(_PTS // _LANES):
            pvec = gr * _LANES + lane
            cvec = jnp.full((_LANES,), 0, jnp.int32) + c
            plsc.store_scatter(idx_v, [pvec, cvec], bvecs[gr] + coff)
        return carry

    lax.fori_loop(0, _C, build, 0)

    def gather_chunk(i, carry):
        pb = i * _GCHUNK
        descs = []
        for j in range(_GCHUNK):
            p = pb + j
            descs.append(pltpu.async_copy(inflat.at[idx_v.at[p]], g_v.at[p], sem))
        for d in descs:
            d.wait()
        return carry

    lax.fori_loop(0, _PTS // _GCHUNK, gather_chunk, 0)

    pltpu.sync_copy(g_v, out_hbm.at[pl.ds(base, _PTS), :])


@functools.partial(
    pl.kernel,
    out_type=jax.ShapeDtypeStruct((_NPAD, _C), jnp.float32),
    mesh=plsc.VectorSubcoreMesh(core_axis_name="c", subcore_axis_name="s"),
    scratch_types=[
        pltpu.VMEM((_PTS,), jnp.int32),
        pltpu.VMEM((_PTS,), jnp.int32),
        pltpu.VMEM((_PTS,), jnp.int32),
        pltpu.VMEM((_PTS, _C), jnp.int32),
        pltpu.VMEM((_PTS, _C), jnp.float32),
        pltpu.SemaphoreType.DMA,
    ],
)
def _sc_gather(*args):
    _sc_gather_kernel(*args)


def _bcast_body(g_ref, out_ref):
    out_ref[...] = jnp.broadcast_to(g_ref[...][:, :, None], out_ref.shape)


def _tc_broadcast(g):
    return pl.pallas_call(
        _bcast_body,
        grid=(_N // _BN,),
        in_specs=[pl.BlockSpec((_BN, _C), lambda i: (i, 0))],
        out_specs=pl.BlockSpec((_BN, _C, _W), lambda i: (i, 0, 0)),
        out_shape=jax.ShapeDtypeStruct((_N, _C, _W), jnp.float32),
    )(g)


def kernel(input, image_ids, centers):
    pad = _NPAD - _N
    ids = jnp.pad(image_ids.astype(jnp.int32), (0, pad))
    rows = jnp.pad(centers[:, 0].astype(jnp.int32), (0, pad))
    cols = jnp.pad(centers[:, 1].astype(jnp.int32), (0, pad))
    g = _sc_gather(input.reshape(-1), ids, rows, cols)
    return _tc_broadcast(g)


# speedup vs baseline: 1.5628x; 1.5628x over previous
import functools

import jax
import jax.numpy as jnp
from jax import lax
from jax.experimental import pallas as pl
from jax.experimental.pallas import tpu as pltpu
from jax.experimental.pallas import tpu_sc as plsc

_B, _C, _H, _W = 8, 96, 224, 224
_N = 2000
_HW = _H * _W
_CHW = _C * _HW

_NUM_WORKERS = 32
_NPAD = 2048
_PTS = _NPAD // _NUM_WORKERS
_LANES = 16
_GCHUNK = 8


def _sc_gather_kernel(inflat, ids_hbm, rows_hbm, cols_hbm, out_hbm,
                      ids_v, rows_v, cols_v, idx_v, g_v, sem):
    nc = lax.axis_size("c")
    wid = lax.axis_index("s") * nc + lax.axis_index("c")
    base = wid * _PTS

    pltpu.sync_copy(ids_hbm.at[pl.ds(base, _PTS)], ids_v)
    pltpu.sync_copy(rows_hbm.at[pl.ds(base, _PTS)], rows_v)
    pltpu.sync_copy(cols_hbm.at[pl.ds(base, _PTS)], cols_v)

    bvecs = []
    for gr in range(_PTS // _LANES):
        sl = pl.ds(gr * _LANES, _LANES)
        bvecs.append(ids_v[sl] * _CHW + rows_v[sl] * _W + cols_v[sl])

    def build(c, carry):
        coff = c * _HW
        for gr in range(_PTS // _LANES):
            idx_v[pl.ds(c * _PTS + gr * _LANES, _LANES)] = bvecs[gr] + coff
        return carry

    lax.fori_loop(0, _C, build, 0)

    def gather_chunk(i, carry):
        cb = i * _GCHUNK
        descs = []
        for j in range(_GCHUNK):
            c = cb + j
            descs.append(pltpu.async_copy(
                inflat.at[idx_v.at[pl.ds(c * _PTS, _PTS)]], g_v.at[c], sem))
        for d in descs:
            d.wait()
        return carry

    lax.fori_loop(0, _C // _GCHUNK, gather_chunk, 0)

    pltpu.sync_copy(g_v, out_hbm.at[wid])


@functools.cache
def _sc_gather():
    return pl.kernel(
        _sc_gather_kernel,
        out_type=jax.ShapeDtypeStruct((_NUM_WORKERS, _C, _PTS), jnp.float32),
        mesh=plsc.VectorSubcoreMesh(
            core_axis_name="c", subcore_axis_name="s",
            num_cores=2, num_subcores=16,
        ),
        scratch_types=[
            pltpu.VMEM((_PTS,), jnp.int32),
            pltpu.VMEM((_PTS,), jnp.int32),
            pltpu.VMEM((_PTS,), jnp.int32),
            pltpu.VMEM((_C * _PTS,), jnp.int32),
            pltpu.VMEM((_C, _PTS), jnp.float32),
            pltpu.SemaphoreType.DMA,
        ],
    )


def _bcast_body(g_ref, out_ref):
    g = g_ref[0]
    out_ref[...] = jnp.broadcast_to(g.T[:, :, None], out_ref.shape)


def _tc_broadcast(g):
    return pl.pallas_call(
        _bcast_body,
        grid=(_NUM_WORKERS,),
        in_specs=[pl.BlockSpec((1, _C, _PTS), lambda i: (i, 0, 0))],
        out_specs=pl.BlockSpec((_PTS, _C, _W), lambda i: (i, 0, 0)),
        out_shape=jax.ShapeDtypeStruct((_N, _C, _W), jnp.float32),
    )(g)


def kernel(input, image_ids, centers):
    pad = _NPAD - _N
    ids = jnp.pad(image_ids.astype(jnp.int32), (0, pad))
    rows = jnp.pad(centers[:, 0].astype(jnp.int32), (0, pad))
    cols = jnp.pad(centers[:, 1].astype(jnp.int32), (0, pad))
    g = _sc_gather()(input.reshape(-1), ids, rows, cols)
    return _tc_broadcast(g)
